# baseline (speedup 1.0000x reference)
import jax
import jax.numpy as jnp
from jax.experimental import pallas as pl
from jax.experimental.pallas import tpu as pltpu

_N = 100000
_D = 128
_H = 128
_B = 64
_R = 5000
_NBLK = _N // _R

_PREC = jax.lax.Precision.DEFAULT


def _body(ids_ref, feat_ref, wg_ref, wf_ref, bf_ref,
          out_ref, m_ref, s_ref, v_ref):
    i = pl.program_id(0)
    nb = pl.num_programs(0)

    @pl.when(i == 0)
    def _init():
        m_ref[...] = jnp.full_like(m_ref, -jnp.inf)
        s_ref[...] = jnp.zeros_like(s_ref)
        v_ref[...] = jnp.zeros_like(v_ref)

    feat = feat_ref[...]
    featb = feat.astype(jnp.bfloat16)
    ids = ids_ref[0, 0, :]

    g = jax.lax.dot_general(wg_ref[...], featb, (((1,), (1,)), ((), ())),
                            preferred_element_type=jnp.float32,
                            precision=_PREC)
    mb = jnp.max(g)
    e = jnp.exp(g - mb)
    e16 = e.astype(jnp.bfloat16)

    ids16 = ids.astype(jnp.int16)
    cmp = (jax.lax.broadcasted_iota(jnp.int16, (_B, _R), 0)
           == ids16[None, :])
    web = jnp.where(cmp, e16, jnp.bfloat16(0.0))
    s_blk = jnp.sum(web.astype(jnp.float32), axis=1, keepdims=True)

    present = s_blk > 0.0
    m_old = m_ref[...]
    m_cand = jnp.where(present, mb, -jnp.inf)
    m_new = jnp.maximum(m_old, m_cand)
    scale_old = jnp.where(m_old == -jnp.inf, 0.0, jnp.exp(m_old - m_new))
    scale_blk = jnp.where(present, jnp.exp(mb - m_new), 0.0)

    v_blk = jax.lax.dot_general(web, featb, (((1,), (0,)), ((), ())),
                                preferred_element_type=jnp.float32,
                                precision=_PREC)
    s_ref[...] = s_ref[...] * scale_old + s_blk * scale_blk
    v_ref[...] = v_ref[...] * scale_old + v_blk * scale_blk
    m_ref[...] = m_new

    @pl.when(i == nb - 1)
    def _finish():
        s = s_ref[...]
        pooled = jnp.where(s > 0, v_ref[...] / jnp.where(s > 0, s, 1.0), 0.0)
        out_ref[...] = jax.lax.dot_general(
            pooled, wf_ref[...], (((1,), (0,)), ((), ())),
            preferred_element_type=jnp.float32,
            precision=jax.lax.Precision.HIGHEST) + bf_ref[...]


def kernel(feat, segment_ids, W_gate, W_feat, b_feat):
    ids3 = segment_ids.reshape(_NBLK, 1, _R)
    wg2 = W_gate.reshape(1, _D).astype(jnp.bfloat16)
    bf2 = b_feat.reshape(1, _H)
    return pl.pallas_call(
        _body,
        grid=(_NBLK,),
        in_specs=[
            pl.BlockSpec((1, 1, _R), lambda i: (i, 0, 0)),
            pl.BlockSpec((_R, _D), lambda i: (i, 0)),
            pl.BlockSpec((1, _D), lambda i: (0, 0)),
            pl.BlockSpec((_D, _H), lambda i: (0, 0)),
            pl.BlockSpec((1, _H), lambda i: (0, 0)),
        ],
        out_specs=pl.BlockSpec((_B, _H), lambda i: (0, 0)),
        out_shape=jax.ShapeDtypeStruct((_B, _H), jnp.float32),
        scratch_shapes=[
            pltpu.VMEM((_B, 1), jnp.float32),
            pltpu.VMEM((_B, 1), jnp.float32),
            pltpu.VMEM((_B, _H), jnp.float32),
        ],
        compiler_params=pltpu.CompilerParams(
            dimension_semantics=("arbitrary",),
        ),
    )(ids3, feat, wg2, W_feat, bf2)


# speedup vs baseline: 1.3263x; 1.3263x over previous
import jax
import jax.numpy as jnp
from jax.experimental import pallas as pl
from jax.experimental.pallas import tpu as pltpu

_N = 100000
_D = 128
_H = 128
_B = 64
_R = 20000
_NBLK = _N // _R

_PREC = jax.lax.Precision.DEFAULT


def _body(ids_ref, feat_ref, wg_ref, wf_ref, bf_ref,
          out_ref, m_ref, s_ref, v_ref):
    i = pl.program_id(0)
    nb = pl.num_programs(0)

    @pl.when(i == 0)
    def _init():
        m_ref[...] = jnp.full_like(m_ref, -jnp.inf)
        s_ref[...] = jnp.zeros_like(s_ref)
        v_ref[...] = jnp.zeros_like(v_ref)

    feat = feat_ref[...]
    featb = feat.astype(jnp.bfloat16)
    ids = ids_ref[0, 0, :]

    g = jax.lax.dot_general(wg_ref[...], featb, (((1,), (1,)), ((), ())),
                            preferred_element_type=jnp.float32,
                            precision=_PREC)
    mb = jnp.max(g)
    e = jnp.exp(g - mb)
    e16 = e.astype(jnp.bfloat16)

    ids16 = ids.astype(jnp.int16)
    cmp = (jax.lax.broadcasted_iota(jnp.int16, (_B, _R), 0)
           == ids16[None, :])
    web = jnp.where(cmp, e16, jnp.bfloat16(0.0))
    s_blk = jnp.sum(web.astype(jnp.float32), axis=1, keepdims=True)

    present = s_blk > 0.0
    m_old = m_ref[...]
    m_cand = jnp.where(present, mb, -jnp.inf)
    m_new = jnp.maximum(m_old, m_cand)
    scale_old = jnp.where(m_old == -jnp.inf, 0.0, jnp.exp(m_old - m_new))
    scale_blk = jnp.where(present, jnp.exp(mb - m_new), 0.0)

    v_blk = jax.lax.dot_general(web, featb, (((1,), (0,)), ((), ())),
                                preferred_element_type=jnp.float32,
                                precision=_PREC)
    s_ref[...] = s_ref[...] * scale_old + s_blk * scale_blk
    v_ref[...] = v_ref[...] * scale_old + v_blk * scale_blk
    m_ref[...] = m_new

    @pl.when(i == nb - 1)
    def _finish():
        s = s_ref[...]
        pooled = jnp.where(s > 0, v_ref[...] / jnp.where(s > 0, s, 1.0), 0.0)
        out_ref[...] = jax.lax.dot_general(
            pooled, wf_ref[...], (((1,), (0,)), ((), ())),
            preferred_element_type=jnp.float32,
            precision=jax.lax.Precision.HIGHEST) + bf_ref[...]


def kernel(feat, segment_ids, W_gate, W_feat, b_feat):
    ids3 = segment_ids.reshape(_NBLK, 1, _R)
    wg2 = W_gate.reshape(1, _D).astype(jnp.bfloat16)
    bf2 = b_feat.reshape(1, _H)
    return pl.pallas_call(
        _body,
        grid=(_NBLK,),
        in_specs=[
            pl.BlockSpec((1, 1, _R), lambda i: (i, 0, 0)),
            pl.BlockSpec((_R, _D), lambda i: (i, 0)),
            pl.BlockSpec((1, _D), lambda i: (0, 0)),
            pl.BlockSpec((_D, _H), lambda i: (0, 0)),
            pl.BlockSpec((1, _H), lambda i: (0, 0)),
        ],
        out_specs=pl.BlockSpec((_B, _H), lambda i: (0, 0)),
        out_shape=jax.ShapeDtypeStruct((_B, _H), jnp.float32),
        scratch_shapes=[
            pltpu.VMEM((_B, 1), jnp.float32),
            pltpu.VMEM((_B, 1), jnp.float32),
            pltpu.VMEM((_B, _H), jnp.float32),
        ],
        compiler_params=pltpu.CompilerParams(
            dimension_semantics=("arbitrary",),
        ),
    )(ids3, feat, wg2, W_feat, bf2)


# speedup vs baseline: 1.3455x; 1.0145x over previous
import jax
import jax.numpy as jnp
from jax.experimental import pallas as pl
from jax.experimental.pallas import tpu as pltpu

_N = 100000
_D = 128
_H = 128
_B = 64
_R = 25000
_NBLK = _N // _R

_PREC = jax.lax.Precision.DEFAULT


def _body(ids_ref, feat_ref, wg_ref, wf_ref, bf_ref,
          out_ref, m_ref, s_ref, v_ref):
    i = pl.program_id(0)
    nb = pl.num_programs(0)

    @pl.when(i == 0)
    def _init():
        m_ref[...] = jnp.full_like(m_ref, -jnp.inf)
        s_ref[...] = jnp.zeros_like(s_ref)
        v_ref[...] = jnp.zeros_like(v_ref)

    feat = feat_ref[...]
    featb = feat.astype(jnp.bfloat16)
    ids = ids_ref[0, 0, :]

    g = jax.lax.dot_general(wg_ref[...], featb, (((1,), (1,)), ((), ())),
                            preferred_element_type=jnp.float32,
                            precision=_PREC)
    mb = jnp.max(g)
    e = jnp.exp(g - mb)
    e16 = e.astype(jnp.bfloat16)

    ids16 = ids.astype(jnp.int16)
    cmp = (jax.lax.broadcasted_iota(jnp.int16, (_B, _R), 0)
           == ids16[None, :])
    web = jnp.where(cmp, e16, jnp.bfloat16(0.0))
    s_blk = jnp.sum(web.astype(jnp.float32), axis=1, keepdims=True)

    present = s_blk > 0.0
    m_old = m_ref[...]
    m_cand = jnp.where(present, mb, -jnp.inf)
    m_new = jnp.maximum(m_old, m_cand)
    scale_old = jnp.where(m_old == -jnp.inf, 0.0, jnp.exp(m_old - m_new))
    scale_blk = jnp.where(present, jnp.exp(mb - m_new), 0.0)

    v_blk = jax.lax.dot_general(web, featb, (((1,), (0,)), ((), ())),
                                preferred_element_type=jnp.float32,
                                precision=_PREC)
    s_ref[...] = s_ref[...] * scale_old + s_blk * scale_blk
    v_ref[...] = v_ref[...] * scale_old + v_blk * scale_blk
    m_ref[...] = m_new

    @pl.when(i == nb - 1)
    def _finish():
        s = s_ref[...]
        pooled = jnp.where(s > 0, v_ref[...] / jnp.where(s > 0, s, 1.0), 0.0)
        out_ref[...] = jax.lax.dot_general(
            pooled, wf_ref[...], (((1,), (0,)), ((), ())),
            preferred_element_type=jnp.float32,
            precision=jax.lax.Precision.HIGHEST) + bf_ref[...]


def kernel(feat, segment_ids, W_gate, W_feat, b_feat):
    ids3 = segment_ids.reshape(_NBLK, 1, _R)
    wg2 = W_gate.reshape(1, _D).astype(jnp.bfloat16)
    bf2 = b_feat.reshape(1, _H)
    return pl.pallas_call(
        _body,
        grid=(_NBLK,),
        in_specs=[
            pl.BlockSpec((1, 1, _R), lambda i: (i, 0, 0)),
            pl.BlockSpec((_R, _D), lambda i: (i, 0)),
            pl.BlockSpec((1, _D), lambda i: (0, 0)),
            pl.BlockSpec((_D, _H), lambda i: (0, 0)),
            pl.BlockSpec((1, _H), lambda i: (0, 0)),
        ],
        out_specs=pl.BlockSpec((_B, _H), lambda i: (0, 0)),
        out_shape=jax.ShapeDtypeStruct((_B, _H), jnp.float32),
        scratch_shapes=[
            pltpu.VMEM((_B, 1), jnp.float32),
            pltpu.VMEM((_B, 1), jnp.float32),
            pltpu.VMEM((_B, _H), jnp.float32),
        ],
        compiler_params=pltpu.CompilerParams(
            dimension_semantics=("arbitrary",),
        ),
    )(ids3, feat, wg2, W_feat, bf2)
